# baseline (device time: 158283 ns/iter reference)
import jax
import jax.numpy as jnp
from jax import lax
from jax.experimental import pallas as pl
from jax.experimental.pallas import tpu as pltpu

N_DEV = 4
M_PER = 1024
D = 1024
TILE = 512


def kernel(x, W1, W2):
    xb = x.astype(jnp.bfloat16)
    w1b = W1.astype(jnp.bfloat16)
    w2b = W2.astype(jnp.bfloat16)

    def body(x_ref, w1_ref, w2_ref, out_ref, xg, csend, prec,
             ag_ssem, ag_rsem, rs_ssem, rs_rsem):
        i = lax.axis_index("i")

        barrier = pltpu.get_barrier_semaphore()
        for d in (1, 2, 3):
            pl.semaphore_signal(
                barrier, inc=1,
                device_id=((i + d) % N_DEV,),
                device_id_type=pl.DeviceIdType.MESH,
            )
        pl.semaphore_wait(barrier, 3)

        ag = []
        for d in (1, 2, 3):
            r = pltpu.make_async_remote_copy(
                src_ref=x_ref,
                dst_ref=xg.at[3 - d],
                send_sem=ag_ssem.at[d - 1],
                recv_sem=ag_rsem.at[3 - d],
                device_id=((i + d) % N_DEV,),
                device_id_type=pl.DeviceIdType.MESH,
            )
            r.start()
            ag.append(r)

        def compute(src, emit):
            for t0 in range(0, M_PER, TILE):
                h = jnp.dot(src[t0:t0 + TILE, :], w1_ref[...],
                            preferred_element_type=jnp.float32)
                h = h * jax.nn.sigmoid(h)
                c = jnp.dot(h.astype(jnp.bfloat16), w2_ref[...],
                            preferred_element_type=jnp.float32)
                emit(t0, c)

        def to_out(t0, c):
            out_ref[t0:t0 + TILE, :] = c
        compute(x_ref[...], to_out)

        rs = []
        for s in (0, 2, 1):
            ag[2 - s].wait()
            def to_csend(t0, c, s=s):
                csend[s, t0:t0 + TILE, :] = c.astype(jnp.bfloat16)
            compute(xg[s], to_csend)
            r = pltpu.make_async_remote_copy(
                src_ref=csend.at[s],
                dst_ref=prec.at[s],
                send_sem=rs_ssem.at[s],
                recv_sem=rs_rsem.at[s],
                device_id=((i + s + 1) % N_DEV,),
                device_id_type=pl.DeviceIdType.MESH,
            )
            r.start()
            rs.append(r)

        for r in rs:
            r.wait()
        out_ref[...] = (out_ref[...]
                        + prec[0].astype(jnp.float32)
                        + prec[1].astype(jnp.float32)
                        + prec[2].astype(jnp.float32))

    return pl.pallas_call(
        body,
        out_shape=jax.ShapeDtypeStruct((M_PER, D), jnp.float32),
        in_specs=[
            pl.BlockSpec(memory_space=pltpu.VMEM),
            pl.BlockSpec(memory_space=pltpu.VMEM),
            pl.BlockSpec(memory_space=pltpu.VMEM),
        ],
        out_specs=pl.BlockSpec(memory_space=pltpu.VMEM),
        scratch_shapes=[
            pltpu.VMEM((3, M_PER, D), jnp.bfloat16),
            pltpu.VMEM((3, M_PER, D), jnp.bfloat16),
            pltpu.VMEM((3, M_PER, D), jnp.bfloat16),
            pltpu.SemaphoreType.DMA((3,)),
            pltpu.SemaphoreType.DMA((3,)),
            pltpu.SemaphoreType.DMA((3,)),
            pltpu.SemaphoreType.DMA((3,)),
        ],
        compiler_params=pltpu.CompilerParams(collective_id=0),
    )(xb, w1b, w2b)


# device time: 128574 ns/iter; 1.2311x vs baseline; 1.2311x over previous
import jax
import jax.numpy as jnp
from jax import lax
from jax.experimental import pallas as pl
from jax.experimental.pallas import tpu as pltpu

N_DEV = 4
M_PER = 1024
D = 1024
F = 4096
TILE = 256


def kernel(x, W1, W2):
    xb = x.astype(jnp.bfloat16)

    def body(x_ref, w1_hbm, w2_hbm, out_ref, xg, prec, w1b, w2b,
             stw1, stw2, ag_ssem, ag_rsem, rs_ssem, rs_rsem, w_sem):
        i = lax.axis_index("i")

        barrier = pltpu.get_barrier_semaphore()
        for d in (1, 2, 3):
            pl.semaphore_signal(
                barrier, inc=1,
                device_id=((i + d) % N_DEV,),
                device_id_type=pl.DeviceIdType.MESH,
            )
        pl.semaphore_wait(barrier, 3)

        def ag_copy(d):
            return pltpu.make_async_remote_copy(
                src_ref=x_ref,
                dst_ref=xg.at[3 - d],
                send_sem=ag_ssem.at[d - 1],
                recv_sem=ag_rsem.at[3 - d],
                device_id=((i + d) % N_DEV,),
                device_id_type=pl.DeviceIdType.MESH,
            )
        ag1, ag2, ag3 = ag_copy(1), ag_copy(2), ag_copy(3)
        ag2.start()

        def stream(src, stage, dst, rows, ntiles):
            def mk(t):
                return pltpu.make_async_copy(
                    src.at[pl.ds(t * rows, rows), :],
                    stage.at[t % 2],
                    w_sem.at[t % 2],
                )
            mk(0).start()

            def step(t, carry):
                mk(t).start()
                mk(t - 1).wait()
                dst[pl.ds((t - 1) * rows, rows), :] = (
                    stage[(t - 1) % 2].astype(jnp.bfloat16))
                return carry
            lax.fori_loop(1, ntiles, step, 0)
            mk(ntiles - 1).wait()
            dst[pl.ds((ntiles - 1) * rows, rows), :] = (
                stage[(ntiles - 1) % 2].astype(jnp.bfloat16))

        stream(w1_hbm, stw1, w1b, 128, 8)
        stream(w2_hbm, stw2, w2b, 256, 16)

        ag2.wait_send()
        ag1.start()
        ag3.start()

        def compute(read, store):
            def step(t, carry):
                t0 = t * TILE
                h = jnp.dot(read(t0), w1b[...],
                            preferred_element_type=jnp.float32)
                h = h * jax.nn.sigmoid(h)
                c = jnp.dot(h.astype(jnp.bfloat16), w2b[...],
                            preferred_element_type=jnp.float32)
                store(t0, c)
                return carry
            lax.fori_loop(0, M_PER // TILE, step, 0)

        rs_list = []
        for s, agd in ((1, ag2), (0, ag3), (2, ag1)):
            agd.wait_recv()
            def read(t0, s=s):
                return xg[s, pl.ds(t0, TILE), :]
            def store(t0, c, s=s):
                xg[s, pl.ds(t0, TILE), :] = c.astype(jnp.bfloat16)
            compute(read, store)
            r = pltpu.make_async_remote_copy(
                src_ref=xg.at[s],
                dst_ref=prec.at[s],
                send_sem=rs_ssem.at[s],
                recv_sem=rs_rsem.at[s],
                device_id=((i + s + 1) % N_DEV,),
                device_id_type=pl.DeviceIdType.MESH,
            )
            r.start()
            rs_list.append(r)

        def read_own(t0):
            return x_ref[pl.ds(t0, TILE), :]
        def to_out(t0, c):
            out_ref[pl.ds(t0, TILE), :] = c
        compute(read_own, to_out)

        for r in rs_list:
            r.wait_recv()
        out_ref[...] = (out_ref[...]
                        + prec[0].astype(jnp.float32)
                        + prec[1].astype(jnp.float32)
                        + prec[2].astype(jnp.float32))

        ag1.wait_send()
        ag3.wait_send()
        for r in rs_list:
            r.wait_send()

    return pl.pallas_call(
        body,
        out_shape=jax.ShapeDtypeStruct((M_PER, D), jnp.float32),
        in_specs=[
            pl.BlockSpec(memory_space=pltpu.VMEM),
            pl.BlockSpec(memory_space=pl.ANY),
            pl.BlockSpec(memory_space=pl.ANY),
        ],
        out_specs=pl.BlockSpec(memory_space=pltpu.VMEM),
        scratch_shapes=[
            pltpu.VMEM((3, M_PER, D), jnp.bfloat16),
            pltpu.VMEM((3, M_PER, D), jnp.bfloat16),
            pltpu.VMEM((D, F), jnp.bfloat16),
            pltpu.VMEM((F, D), jnp.bfloat16),
            pltpu.VMEM((2, 128, F), jnp.float32),
            pltpu.VMEM((2, 256, D), jnp.float32),
            pltpu.SemaphoreType.DMA((3,)),
            pltpu.SemaphoreType.DMA((3,)),
            pltpu.SemaphoreType.DMA((3,)),
            pltpu.SemaphoreType.DMA((3,)),
            pltpu.SemaphoreType.DMA((2,)),
        ],
        compiler_params=pltpu.CompilerParams(
            collective_id=0,
            vmem_limit_bytes=56 * 1024 * 1024,
        ),
    )(xb, W1, W2)


# device time: 107771 ns/iter; 1.4687x vs baseline; 1.1930x over previous
import jax
import jax.numpy as jnp
from jax import lax
from jax.experimental import pallas as pl
from jax.experimental.pallas import tpu as pltpu

N_DEV = 4
M_PER = 1024
D = 1024
F = 4096
TILE = 256
NT = M_PER // TILE


def kernel(x, W1, W2):
    xb = x.astype(jnp.bfloat16)

    def body(x_ref, w1_hbm, w2_hbm, out_ref, xg, prec, w1b, w2b,
             stw1, stw2, ag_ssem, ag_rsem, rs_ssem, rs_rsem, w_sem):
        i = lax.axis_index("i")

        barrier = pltpu.get_barrier_semaphore()
        for d in (1, 2, 3):
            pl.semaphore_signal(
                barrier, inc=1,
                device_id=((i + d) % N_DEV,),
                device_id_type=pl.DeviceIdType.MESH,
            )
        pl.semaphore_wait(barrier, 3)

        def ag_copy(d, t):
            return pltpu.make_async_remote_copy(
                src_ref=x_ref.at[pl.ds(t * TILE, TILE), :],
                dst_ref=xg.at[3 - d, pl.ds(t * TILE, TILE), :],
                send_sem=ag_ssem.at[d - 1, t],
                recv_sem=ag_rsem.at[3 - d, t],
                device_id=((i + d) % N_DEV,),
                device_id_type=pl.DeviceIdType.MESH,
            )

        def rs_copy(s, t):
            return pltpu.make_async_remote_copy(
                src_ref=xg.at[s, pl.ds(t * TILE, TILE), :],
                dst_ref=prec.at[s, pl.ds(t * TILE, TILE), :],
                send_sem=rs_ssem.at[s, t],
                recv_sem=rs_rsem.at[s, t],
                device_id=((i + s + 1) % N_DEV,),
                device_id_type=pl.DeviceIdType.MESH,
            )

        for t in range(NT):
            ag_copy(2, t).start()

        def stream(src, stage, dst, rows, ntiles):
            def mk(t):
                return pltpu.make_async_copy(
                    src.at[pl.ds(t * rows, rows), :],
                    stage.at[t % 2],
                    w_sem.at[t % 2],
                )
            mk(0).start()

            def step(t, carry):
                mk(t).start()
                mk(t - 1).wait()
                dst[pl.ds((t - 1) * rows, rows), :] = (
                    stage[(t - 1) % 2].astype(jnp.bfloat16))
                return carry
            lax.fori_loop(1, ntiles, step, 0)
            mk(ntiles - 1).wait()
            dst[pl.ds((ntiles - 1) * rows, rows), :] = (
                stage[(ntiles - 1) % 2].astype(jnp.bfloat16))

        stream(w1_hbm, stw1, w1b, 128, 8)

        for t in range(NT):
            ag_copy(1, t).start()
            ag_copy(3, t).start()

        stream(w2_hbm, stw2, w2b, 256, 16)

        def gemm(xs):
            h = jnp.dot(xs, w1b[...], preferred_element_type=jnp.float32)
            h = h * jax.nn.sigmoid(h)
            return jnp.dot(h.astype(jnp.bfloat16), w2b[...],
                           preferred_element_type=jnp.float32)

        for s, d in ((1, 2), (0, 3), (2, 1)):
            def step(t, carry, s=s, d=d):
                t0 = t * TILE
                ag_copy(d, t).wait_recv()
                c = gemm(xg[s, pl.ds(t0, TILE), :])
                xg[s, pl.ds(t0, TILE), :] = c.astype(jnp.bfloat16)
                rs_copy(s, t).start()
                return carry
            lax.fori_loop(0, NT, step, 0)

        def own_step(t, carry):
            t0 = t * TILE
            out_ref[pl.ds(t0, TILE), :] = gemm(x_ref[pl.ds(t0, TILE), :])
            return carry
        lax.fori_loop(0, NT, own_step, 0)

        for s in range(3):
            def wrecv(t, carry, s=s):
                rs_copy(s, t).wait_recv()
                return carry
            lax.fori_loop(0, NT, wrecv, 0)
        out_ref[...] = (out_ref[...]
                        + prec[0].astype(jnp.float32)
                        + prec[1].astype(jnp.float32)
                        + prec[2].astype(jnp.float32))

        for d in (1, 2, 3):
            def wsend_ag(t, carry, d=d):
                ag_copy(d, t).wait_send()
                return carry
            lax.fori_loop(0, NT, wsend_ag, 0)
        for s in range(3):
            def wsend_rs(t, carry, s=s):
                rs_copy(s, t).wait_send()
                return carry
            lax.fori_loop(0, NT, wsend_rs, 0)

    return pl.pallas_call(
        body,
        out_shape=jax.ShapeDtypeStruct((M_PER, D), jnp.float32),
        in_specs=[
            pl.BlockSpec(memory_space=pltpu.VMEM),
            pl.BlockSpec(memory_space=pl.ANY),
            pl.BlockSpec(memory_space=pl.ANY),
        ],
        out_specs=pl.BlockSpec(memory_space=pltpu.VMEM),
        scratch_shapes=[
            pltpu.VMEM((3, M_PER, D), jnp.bfloat16),
            pltpu.VMEM((3, M_PER, D), jnp.bfloat16),
            pltpu.VMEM((D, F), jnp.bfloat16),
            pltpu.VMEM((F, D), jnp.bfloat16),
            pltpu.VMEM((2, 128, F), jnp.float32),
            pltpu.VMEM((2, 256, D), jnp.float32),
            pltpu.SemaphoreType.DMA((3, NT)),
            pltpu.SemaphoreType.DMA((3, NT)),
            pltpu.SemaphoreType.DMA((3, NT)),
            pltpu.SemaphoreType.DMA((3, NT)),
            pltpu.SemaphoreType.DMA((2,)),
        ],
        compiler_params=pltpu.CompilerParams(
            collective_id=0,
            vmem_limit_bytes=56 * 1024 * 1024,
        ),
    )(xb, W1, W2)
